# Initial kernel scaffold; baseline (speedup 1.0000x reference)
import jax
import jax.numpy as jnp
from jax.experimental import pallas as pl


def kernel(x, tok_table, pos_table):
    raise NotImplementedError("write your pallas kernel here")


# speedup vs baseline: 3.5882x; 3.5882x over previous
import functools

import jax
import jax.numpy as jnp
from jax import lax
from jax.experimental import pallas as pl
from jax.experimental.pallas import tpu as pltpu
from jax.experimental.pallas import tpu_sc as plsc

_NUM_WORKERS = 32
_CHUNK = 128


def _add_body(a_ref, b_ref, o_ref):
    o_ref[...] = a_ref[...] + b_ref[...]


def _make_sc_gather(n, d, per_w, steps):
    mesh = plsc.VectorSubcoreMesh(core_axis_name="c", subcore_axis_name="s")

    @functools.partial(
        pl.kernel,
        mesh=mesh,
        out_type=jax.ShapeDtypeStruct((n, d), jnp.float32),
        compiler_params=pltpu.CompilerParams(use_tc_tiling_on_sc=False),
        scratch_types=[
            pltpu.VMEM((_CHUNK,), jnp.int32),
            pltpu.VMEM((_CHUNK, d), jnp.float32),
            pltpu.SemaphoreType.DMA,
        ],
    )
    def sc_gather(comb_hbm, idx_hbm, out_hbm, idx_v, rows_v, sem):
        wid = lax.axis_index("s") * 2 + lax.axis_index("c")
        base = wid * per_w

        def step(j, carry):
            start = base + j * _CHUNK
            pltpu.sync_copy(idx_hbm.at[pl.ds(start, _CHUNK)], idx_v)
            pltpu.async_copy(comb_hbm.at[idx_v], rows_v, sem).wait()
            pltpu.sync_copy(rows_v, out_hbm.at[pl.ds(start, _CHUNK)])
            return carry

        lax.fori_loop(0, steps, step, 0)

    return sc_gather


def kernel(x, tok_table, pos_table):
    b, l = x.shape
    r, d = pos_table.shape
    n = b * l
    per_w = n // _NUM_WORKERS
    steps = per_w // _CHUNK

    xf = x.reshape(n).astype(jnp.int32)

    comb = pl.pallas_call(
        _add_body,
        out_shape=jax.ShapeDtypeStruct((r, d), jnp.float32),
    )(tok_table[:r], pos_table)

    out = _make_sc_gather(n, d, per_w, steps)(comb, xf)
    return out.reshape(b, l, d)


# speedup vs baseline: 3.9352x; 1.0967x over previous
import functools

import jax
import jax.numpy as jnp
from jax import lax
from jax.experimental import pallas as pl
from jax.experimental.pallas import tpu as pltpu
from jax.experimental.pallas import tpu_sc as plsc

_NUM_WORKERS = 32
_CHUNK = 128
_NB = 8
_H = 4


def _add_body(a_ref, b_ref, o_ref):
    o_ref[...] = a_ref[...] + b_ref[...]


def _make_sc_gather(n, d, per_w, steps):
    mesh = plsc.VectorSubcoreMesh(core_axis_name="c", subcore_axis_name="s")

    @functools.partial(
        pl.kernel,
        mesh=mesh,
        out_type=jax.ShapeDtypeStruct((n, d), jnp.float32),
        compiler_params=pltpu.CompilerParams(use_tc_tiling_on_sc=False),
        scratch_types=[
            pltpu.VMEM((steps, _CHUNK), jnp.int32),
            pltpu.VMEM((_NB, _CHUNK, d), jnp.float32),
            pltpu.SemaphoreType.DMA((_NB,)),
            pltpu.SemaphoreType.DMA((_NB,)),
        ],
    )
    def sc_gather(comb_hbm, idx_hbm, out_hbm, idx_v, rows_v, gsem, wsem):
        wid = lax.axis_index("s") * 2 + lax.axis_index("c")
        base = wid * per_w
        pltpu.sync_copy(idx_hbm.at[pl.ds(wid * steps, steps)], idx_v)

        def start_gather(i, b):
            pltpu.async_copy(comb_hbm.at[idx_v.at[i]], rows_v.at[b], gsem.at[b])

        def wait_gather(i, b):
            pltpu.make_async_copy(
                comb_hbm.at[idx_v.at[i]], rows_v.at[b], gsem.at[b]
            ).wait()

        def start_write(t, b):
            pltpu.async_copy(
                rows_v.at[b], out_hbm.at[pl.ds(base + t * _CHUNK, _CHUNK)],
                wsem.at[b],
            )

        def wait_write(t, b):
            pltpu.make_async_copy(
                rows_v.at[b], out_hbm.at[pl.ds(base + t * _CHUNK, _CHUNK)],
                wsem.at[b],
            ).wait()

        for b in range(_NB):
            start_gather(b, b)
            if b >= _H:
                t = b - _H
                wait_gather(t, t % _NB)
                start_write(t, t % _NB)

        def block(jo, carry):
            for b in range(_NB):
                i = jo * _NB + b
                wait_write(i - _NB, b)
                start_gather(i, b)
                t = i - _H
                wait_gather(t, t % _NB)
                start_write(t, t % _NB)
            return carry

        lax.fori_loop(1, steps // _NB, block, 0)

        for i in range(steps, steps + _H):
            t = i - _H
            wait_gather(t, t % _NB)
            start_write(t, t % _NB)
        for t in range(steps - _NB, steps):
            wait_write(t, t % _NB)

    return sc_gather


def kernel(x, tok_table, pos_table):
    b, l = x.shape
    r, d = pos_table.shape
    n = b * l
    per_w = n // _NUM_WORKERS
    steps = per_w // _CHUNK

    xf = x.reshape(n // _CHUNK, _CHUNK).astype(jnp.int32)

    comb = pl.pallas_call(
        _add_body,
        out_shape=jax.ShapeDtypeStruct((r, d), jnp.float32),
    )(tok_table[:r], pos_table)

    out = _make_sc_gather(n, d, per_w, steps)(comb, xf)
    return out.reshape(b, l, d)


# speedup vs baseline: 3.9370x; 1.0005x over previous
import functools

import jax
import jax.numpy as jnp
from jax import lax
from jax.experimental import pallas as pl
from jax.experimental.pallas import tpu as pltpu
from jax.experimental.pallas import tpu_sc as plsc

_NUM_WORKERS = 32
_NB = 4
_H = 2
_SPLIT = 128


def _add_body(a_ref, b_ref, o_ref):
    o_ref[...] = a_ref[...] + b_ref[...]


def _make_sc_gather(b, l, d, rows_per_w):
    mesh = plsc.VectorSubcoreMesh(core_axis_name="c", subcore_axis_name="s")
    seg2 = l - _SPLIT

    @functools.partial(
        pl.kernel,
        mesh=mesh,
        out_type=jax.ShapeDtypeStruct((b, l, d), jnp.float32),
        compiler_params=pltpu.CompilerParams(use_tc_tiling_on_sc=False),
        scratch_types=[
            pltpu.VMEM((rows_per_w, l), jnp.int32),
            pltpu.VMEM((_NB, l, d), jnp.float32),
            pltpu.SemaphoreType.DMA((_NB,)),
            pltpu.SemaphoreType.DMA((_NB,)),
        ],
    )
    def sc_gather(comb_hbm, idx_hbm, out_hbm, idx_v, rows_v, gsem, wsem):
        wid = lax.axis_index("s") * 2 + lax.axis_index("c")
        base = wid * rows_per_w
        pltpu.sync_copy(idx_hbm.at[pl.ds(base, rows_per_w)], idx_v)

        def gather_parts(i, bf):
            yield (comb_hbm.at[idx_v.at[i, pl.ds(0, _SPLIT)]],
                   rows_v.at[bf, pl.ds(0, _SPLIT)], gsem.at[bf])
            yield (comb_hbm.at[idx_v.at[i, pl.ds(_SPLIT, seg2)]],
                   rows_v.at[bf, pl.ds(_SPLIT, seg2)], gsem.at[bf])

        def start_gather(i, bf):
            for src, dst, sem in gather_parts(i, bf):
                pltpu.async_copy(src, dst, sem)

        def wait_gather(i, bf):
            for src, dst, sem in gather_parts(i, bf):
                pltpu.make_async_copy(src, dst, sem).wait()

        def start_write(t, bf):
            pltpu.async_copy(rows_v.at[bf], out_hbm.at[base + t], wsem.at[bf])

        def wait_write(t, bf):
            pltpu.make_async_copy(
                rows_v.at[bf], out_hbm.at[base + t], wsem.at[bf]
            ).wait()

        for bf in range(_NB):
            start_gather(bf, bf)
            if bf >= _H:
                t = bf - _H
                wait_gather(t, t % _NB)
                start_write(t, t % _NB)

        def block(jo, carry):
            for bf in range(_NB):
                i = jo * _NB + bf
                wait_write(i - _NB, bf)
                start_gather(i, bf)
                t = i - _H
                wait_gather(t, t % _NB)
                start_write(t, t % _NB)
            return carry

        lax.fori_loop(1, rows_per_w // _NB, block, 0)

        for i in range(rows_per_w, rows_per_w + _H):
            t = i - _H
            wait_gather(t, t % _NB)
            start_write(t, t % _NB)
        for t in range(rows_per_w - _NB, rows_per_w):
            wait_write(t, t % _NB)

    return sc_gather


def kernel(x, tok_table, pos_table):
    b, l = x.shape
    r, d = pos_table.shape
    rows_per_w = b // _NUM_WORKERS

    xi = x.astype(jnp.int32)

    comb = pl.pallas_call(
        _add_body,
        out_shape=jax.ShapeDtypeStruct((r, d), jnp.float32),
    )(tok_table[:r], pos_table)

    return _make_sc_gather(b, l, d, rows_per_w)(comb, xi)


# speedup vs baseline: 26.3934x; 6.7040x over previous
import functools

import jax
import jax.numpy as jnp
from jax import lax
from jax.experimental import pallas as pl
from jax.experimental.pallas import tpu as pltpu
from jax.experimental.pallas import tpu_sc as plsc

_NUM_WORKERS = 32
_NB = 4
_LANES = 16


def _add_t_body(a_ref, b_ref, o_ref):
    o_ref[...] = (a_ref[...] + b_ref[...]).T


def _make_sc_gather(b, l, d, r):
    mesh = plsc.VectorSubcoreMesh(core_axis_name="c", subcore_axis_name="s")
    bw = b // _NUM_WORKERS
    dhi, dlo = d // 8, 8
    bhi = b // bw

    @functools.partial(
        pl.kernel,
        mesh=mesh,
        out_type=jax.ShapeDtypeStruct((l, dhi, bhi, dlo, bw), jnp.float32),
        compiler_params=pltpu.CompilerParams(
            use_tc_tiling_on_sc=False, needs_layout_passes=False
        ),
        scratch_types=[
            pltpu.VMEM((d * r,), jnp.float32),
            pltpu.VMEM((l, bw), jnp.int32),
            pltpu.VMEM((_NB, dhi, dlo, bw), jnp.float32),
            pltpu.SemaphoreType.DMA((_NB,)),
        ],
    )
    def sc_gather(combt_hbm, xt_hbm, out_hbm, combt_v, idxt_v, tiles_v, wsem):
        wid = lax.axis_index("s") * 2 + lax.axis_index("c")
        b0 = wid * bw
        pltpu.sync_copy(combt_hbm, combt_v)
        pltpu.sync_copy(xt_hbm.at[:, pl.ds(b0, bw)], idxt_v)

        def build_tile(li, bf):
            ivs = [idxt_v[li, pl.ds(j * _LANES, _LANES)] for j in range(bw // _LANES)]

            @plsc.parallel_loop(0, d, 1, unroll=4)
            def dd_body(dd):
                row = combt_v.at[pl.ds(dd * r, r)]
                for j, iv in enumerate(ivs):
                    vals = plsc.load_gather(row, [iv])
                    tiles_v[bf, dd // dlo, dd % dlo, pl.ds(j * _LANES, _LANES)] = vals

        def start_write(li, bf):
            pltpu.async_copy(tiles_v.at[bf], out_hbm.at[li, :, wid], wsem.at[bf])

        def wait_write(li, bf):
            pltpu.make_async_copy(
                tiles_v.at[bf], out_hbm.at[li, :, wid], wsem.at[bf]
            ).wait()

        for bf in range(_NB):
            build_tile(bf, bf)
            start_write(bf, bf)

        def block(l0, carry):
            for bf in range(_NB):
                li = l0 * _NB + bf
                wait_write(li - _NB, bf)
                build_tile(li, bf)
                start_write(li, bf)
            return carry

        lax.fori_loop(1, l // _NB, block, 0)

        for li in range(l - _NB, l):
            wait_write(li, li % _NB)

    return sc_gather


def kernel(x, tok_table, pos_table):
    b, l = x.shape
    r, d = pos_table.shape

    xt = x.T.astype(jnp.int32)

    combt = pl.pallas_call(
        _add_t_body,
        out_shape=jax.ShapeDtypeStruct((d, r), jnp.float32),
    )(tok_table[:r], pos_table)

    out4 = _make_sc_gather(b, l, d, r)(combt.reshape(d * r), xt)
    return out4.transpose(2, 4, 0, 1, 3).reshape(b, l, d)
